# baseline (speedup 1.0000x reference)
import jax
import jax.numpy as jnp
from jax.experimental import pallas as pl

DIM = 256
BLK_MAJ = 8
BLK_R = 1024


def _body(ys_ref, m_ref, w_ref, b_ref, mt_ref, y_ref, yq_ref):
    ys = ys_ref[...]
    m = m_ref[...]
    w = w_ref[0, :]
    b = b_ref[0, :]
    y = (ys[:, :, None] * w[None, None, :] + b[None, None, :]) * m[:, :, None]
    y_ref[...] = y
    yq_ref[...] = jnp.broadcast_to(mt_ref[0, :], y.shape)


def kernel(y_support, padding_obs_support, n_obs_query, W_y, b_y, mask_table):
    batch, n_sup = y_support.shape
    total = batch * n_sup
    R = BLK_R
    G = total // R
    ys2 = y_support.reshape(G, R)
    m2 = jnp.where(padding_obs_support.reshape(G, R), 0.0, 1.0).astype(jnp.float32)
    w2 = W_y.reshape(1, DIM)
    b2 = b_y.reshape(1, DIM)

    y, yq = pl.pallas_call(
        _body,
        grid=(G // BLK_MAJ,),
        in_specs=[
            pl.BlockSpec((BLK_MAJ, R), lambda i: (i, 0)),
            pl.BlockSpec((BLK_MAJ, R), lambda i: (i, 0)),
            pl.BlockSpec((1, DIM), lambda i: (0, 0)),
            pl.BlockSpec((1, DIM), lambda i: (0, 0)),
            pl.BlockSpec((1, DIM), lambda i: (0, 0)),
        ],
        out_specs=[
            pl.BlockSpec((BLK_MAJ, R, DIM), lambda i: (i, 0, 0)),
            pl.BlockSpec((BLK_MAJ, R, DIM), lambda i: (i, 0, 0)),
        ],
        out_shape=[
            jax.ShapeDtypeStruct((G, R, DIM), jnp.float32),
            jax.ShapeDtypeStruct((G, R, DIM), jnp.float32),
        ],
    )(ys2, m2, w2, b2, mask_table)

    return (
        y.reshape(batch, n_sup, DIM),
        yq.reshape(batch, n_sup, 1, DIM),
    )
